# baseline (speedup 1.0000x reference)
import functools

import jax
import jax.numpy as jnp
from jax import lax
from jax.experimental import pallas as pl
from jax.experimental.pallas import tpu as pltpu
from jax.experimental.pallas import tpu_sc as plsc

D_MODEL = 64
SCALE = 8.0
NUM_CORES = 2
NUM_SUBCORES = 16
LANES = 16
NUM_WORKERS = NUM_CORES * NUM_SUBCORES
NBUF = 4


def _embed_lookup(idx, table):
    n_rows, n_cols = idx.shape
    rows_per_w = n_rows // NUM_WORKERS
    assert n_rows % NUM_WORKERS == 0 and rows_per_w % NBUF == 0

    mesh = plsc.VectorSubcoreMesh(core_axis_name="c", subcore_axis_name="s")

    @functools.partial(
        pl.kernel,
        mesh=mesh,
        compiler_params=pltpu.CompilerParams(use_tc_tiling_on_sc=False),
        out_type=jax.ShapeDtypeStruct((n_rows, n_cols, D_MODEL), jnp.float32),
        scratch_types=[
            pltpu.VMEM((rows_per_w, n_cols), jnp.int32),
        ]
        + [pltpu.VMEM((n_cols, D_MODEL), jnp.float32)] * NBUF
        + [pltpu.SemaphoreType.DMA] * (2 * NBUF),
    )
    def k(idx_hbm, table_hbm, out_hbm, idx_v, *bufs_and_sems):
        bufs = bufs_and_sems[:NBUF]
        gsem = bufs_and_sems[NBUF : 2 * NBUF]
        ssem = bufs_and_sems[2 * NBUF :]

        wid = lax.axis_index("s") * NUM_CORES + lax.axis_index("c")
        row0 = wid * rows_per_w
        pltpu.sync_copy(idx_hbm.at[pl.ds(row0, rows_per_w)], idx_v)

        def gather_src(r):
            return table_hbm.at[idx_v.at[r]]

        def out_dst(r):
            return out_hbm.at[row0 + r]

        for b in range(NBUF):
            pltpu.async_copy(gather_src(b), bufs[b], gsem[b])

        @pl.loop(0, rows_per_w, step=NBUF)
        def _(w):
            for b in range(NBUF):
                wb = w + b
                pltpu.make_async_copy(gather_src(wb), bufs[b], gsem[b]).wait()

                @pl.loop(0, n_cols)
                def _(r):
                    for c in range(0, D_MODEL, LANES):
                        slc = (pl.ds(r, 1), pl.ds(c, LANES))
                        bufs[b].at[slc][...] = bufs[b].at[slc][...] * SCALE

                pltpu.async_copy(bufs[b], out_dst(wb), ssem[b])

            for b in range(NBUF):
                wb = w + b
                pltpu.make_async_copy(bufs[b], out_dst(wb), ssem[b]).wait()

                @pl.when(wb + NBUF < rows_per_w)
                def _():
                    pltpu.async_copy(gather_src(wb + NBUF), bufs[b], gsem[b])

    return k(idx, table)


def kernel(x, table):
    return _embed_lookup(x.astype(jnp.int32), table)
